# baseline (speedup 1.0000x reference)
import functools

import jax
import jax.numpy as jnp
from jax import lax
from jax.experimental import pallas as pl
from jax.experimental.pallas import tpu as pltpu
from jax.experimental.pallas import tpu_sc as plsc

_E = 8
_K = 2
_H = 768
_T = 32768
_NCHUNK = 2
_TC = _T // _NCHUNK
_TBLK = 4096

_NW = 32
_TPW = _TC // _NW
_GRP = _TPW // 16


def _logits_block(w_ref, b_ref, hs_ref, out_ref):
    hs = hs_ref[...]
    w = w_ref[...]
    logits = jax.lax.dot_general(
        w, hs, (((1,), (1,)), ((), ())), preferred_element_type=jnp.float32)
    out_ref[...] = logits + b_ref[...]


def _sc_route(logits_hbm, scores_hbm, idx_hbm, lg_v, sc_v, ix_v):
    wid = lax.axis_index("s") * 2 + lax.axis_index("c")
    base = wid * _TPW
    pltpu.sync_copy(logits_hbm.at[:, pl.ds(base, _TPW)], lg_v)

    def group(g, carry):
        l = [lg_v[e, pl.ds(g * 16, 16)] for e in range(_E)]
        m1 = l[0]
        for e in range(1, _E):
            m1 = jnp.maximum(m1, l[e])
        i1 = jnp.where(l[0] == m1, 0, _E)
        for e in range(1, _E):
            i1 = jnp.minimum(i1, jnp.where(l[e] == m1, e, _E))
        neg = jnp.float32(-3.0e38)
        lm = [jnp.where(i1 == e, neg, l[e]) for e in range(_E)]
        m2 = lm[0]
        for e in range(1, _E):
            m2 = jnp.maximum(m2, lm[e])
        i2 = jnp.where(lm[0] == m2, 0, _E)
        for e in range(1, _E):
            i2 = jnp.minimum(i2, jnp.where(lm[e] == m2, e, _E))

        s = jnp.exp(m2 - m1)
        r = 1.0 / (1.0 + s)
        p2 = s * r

        zeros = jnp.zeros((16,), jnp.float32)
        for k in range(_E):
            sc_v[pl.ds(g * 128 + k * 16, 16)] = zeros
        ltok = g * 16 + lax.iota(jnp.int32, 16)
        plsc.store_scatter(sc_v, [ltok * _E + i1], r)
        plsc.store_scatter(sc_v, [ltok * _E + i2], p2)
        plsc.store_scatter(ix_v, [ltok * _K], i1)
        plsc.store_scatter(ix_v, [ltok * _K + 1], i2)
        return carry

    lax.fori_loop(0, _GRP, group, 0)
    pltpu.sync_copy(sc_v, scores_hbm.at[pl.ds(base * _E, _TPW * _E)])
    pltpu.sync_copy(ix_v, idx_hbm.at[pl.ds(base * _K, _TPW * _K)])


_sc_route_call = functools.partial(
    pl.kernel,
    out_type=[
        jax.ShapeDtypeStruct((_TC * _E,), jnp.float32),
        jax.ShapeDtypeStruct((_TC * _K,), jnp.int32),
    ],
    mesh=plsc.VectorSubcoreMesh(
        core_axis_name="c", subcore_axis_name="s",
        num_cores=2, num_subcores=16),
    scratch_types=[
        pltpu.VMEM((_E, _TPW), jnp.float32),
        pltpu.VMEM((_TPW * _E,), jnp.float32),
        pltpu.VMEM((_TPW * _K,), jnp.int32),
    ],
    compiler_params=pltpu.CompilerParams(needs_layout_passes=False),
)(_sc_route)


def _tc_logits(hidden_chunk, router_weight, bias2d):
    grid = (_TC // _TBLK,)
    return pl.pallas_call(
        _logits_block,
        grid=grid,
        in_specs=[
            pl.BlockSpec((_E, _H), lambda i: (0, 0)),
            pl.BlockSpec((_E, 1), lambda i: (0, 0)),
            pl.BlockSpec((_TBLK, _H), lambda i: (i, 0)),
        ],
        out_specs=pl.BlockSpec((_E, _TBLK), lambda i: (0, i)),
        out_shape=jax.ShapeDtypeStruct((_E, _TC), jnp.float32),
        compiler_params=pltpu.CompilerParams(
            dimension_semantics=("parallel",)),
    )(router_weight, bias2d, hidden_chunk)


@jax.jit
def kernel(hidden_states, router_weight, router_bias):
    b2 = router_bias.reshape(_E, 1)
    lg = [_tc_logits(hidden_states[c * _TC:(c + 1) * _TC], router_weight, b2)
          for c in range(_NCHUNK)]
    outs = [_sc_route_call(lg[c]) for c in range(_NCHUNK)]
    scores = jnp.concatenate([o[0] for o in outs]).reshape(_T, _E)
    idx = jnp.concatenate([o[1] for o in outs]).reshape(_T, _K)
    return scores, idx


# speedup vs baseline: 5.0020x; 5.0020x over previous
import jax
import jax.numpy as jnp
from jax.experimental import pallas as pl
from jax.experimental.pallas import tpu as pltpu

_E = 8
_K = 2
_H = 768
_TBLK = 4096


def _router_block(w_ref, b_ref, hs_ref, scores_ref, idx_ref):
    hs = hs_ref[...]
    w = w_ref[...]
    logits = jax.lax.dot_general(
        w, hs, (((1,), (1,)), ((), ())), preferred_element_type=jnp.float32)
    logits = logits + b_ref[...]

    e_iota = jax.lax.broadcasted_iota(jnp.int32, logits.shape, 0)
    m1 = jnp.max(logits, axis=0, keepdims=True)
    i1 = jnp.min(jnp.where(logits == m1, e_iota, _E), axis=0, keepdims=True)
    masked = jnp.where(e_iota == i1, -jnp.inf, logits)
    m2 = jnp.max(masked, axis=0, keepdims=True)
    i2 = jnp.min(jnp.where(masked == m2, e_iota, _E), axis=0, keepdims=True)

    s = jnp.exp(m2 - m1)
    r = 1.0 / (1.0 + s)
    scores_t = (jnp.where(e_iota == i1, r, 0.0)
                + jnp.where(e_iota == i2, s * r, 0.0))
    scores_ref[...] = scores_t
    idx_ref[...] = jnp.concatenate([i1, i2], axis=0)


@jax.jit
def kernel(hidden_states, router_weight, router_bias):
    t = hidden_states.shape[0]
    grid = (t // _TBLK,)
    scores_p, idx_p = pl.pallas_call(
        _router_block,
        grid=grid,
        in_specs=[
            pl.BlockSpec((_E, _H), lambda i: (0, 0)),
            pl.BlockSpec((_E, 1), lambda i: (0, 0)),
            pl.BlockSpec((_TBLK, _H), lambda i: (i, 0)),
        ],
        out_specs=[
            pl.BlockSpec((_E, _TBLK), lambda i: (0, i)),
            pl.BlockSpec((_K, _TBLK), lambda i: (0, i)),
        ],
        out_shape=[
            jax.ShapeDtypeStruct((_E, t), jnp.float32),
            jax.ShapeDtypeStruct((_K, t), jnp.int32),
        ],
        compiler_params=pltpu.CompilerParams(
            dimension_semantics=("parallel",)),
    )(router_weight, router_bias.reshape(_E, 1), hidden_states)
    return scores_p.T, idx_p.T


# speedup vs baseline: 5.0209x; 1.0038x over previous
import jax
import jax.numpy as jnp
from jax.experimental import pallas as pl
from jax.experimental.pallas import tpu as pltpu

_E = 8
_K = 2
_H = 768
_TBLK = 4096


def _router_block(w_ref, b_ref, hs_ref, scores_ref, idx_ref):
    hs = hs_ref[...]
    w = w_ref[...]
    logits = jax.lax.dot_general(
        w, hs, (((1,), (1,)), ((), ())), preferred_element_type=jnp.float32)
    logits = logits + b_ref[...]

    e_iota = jax.lax.broadcasted_iota(jnp.int32, logits.shape, 0)
    m1 = jnp.max(logits, axis=0, keepdims=True)
    i1 = jnp.min(jnp.where(logits == m1, e_iota, _E), axis=0, keepdims=True)
    masked = jnp.where(e_iota == i1, -jnp.inf, logits)
    m2 = jnp.max(masked, axis=0, keepdims=True)
    i2 = jnp.min(jnp.where(masked == m2, e_iota, _E), axis=0, keepdims=True)

    s = jnp.exp(m2 - m1)
    r = 1.0 / (1.0 + s)
    scores_t = (jnp.where(e_iota == i1, r, 0.0)
                + jnp.where(e_iota == i2, s * r, 0.0))
    scores_ref[...] = scores_t
    idx_ref[...] = jnp.concatenate([i1, i2], axis=0)[None]


@jax.jit
def kernel(hidden_states, router_weight, router_bias):
    t = hidden_states.shape[0]
    nblk = t // _TBLK
    scores_p, idx_p = pl.pallas_call(
        _router_block,
        grid=(nblk,),
        in_specs=[
            pl.BlockSpec((_E, _H), lambda i: (0, 0)),
            pl.BlockSpec((_E, 1), lambda i: (0, 0)),
            pl.BlockSpec((_TBLK, _H), lambda i: (i, 0)),
        ],
        out_specs=[
            pl.BlockSpec((_E, _TBLK), lambda i: (i, 0)),
            pl.BlockSpec((1, _K, _TBLK), lambda i: (i, 0, 0)),
        ],
        out_shape=[
            jax.ShapeDtypeStruct((nblk * _E, _TBLK), jnp.float32),
            jax.ShapeDtypeStruct((nblk, _K, _TBLK), jnp.int32),
        ],
        compiler_params=pltpu.CompilerParams(
            dimension_semantics=("parallel",)),
    )(router_weight, router_bias.reshape(_E, 1), hidden_states)
    scores = scores_p.reshape(nblk, _E, _TBLK).transpose(0, 2, 1).reshape(t, _E)
    idx = idx_p.transpose(0, 2, 1).reshape(t, _K)
    return scores, idx
